# baseline (speedup 1.0000x reference)
import functools

import numpy as np
import jax
import jax.numpy as jnp
from jax import lax
from jax.experimental import pallas as pl
from jax.experimental.pallas import tpu as pltpu
from jax.experimental.pallas import tpu_sc as plsc

_N = 10000
_E = 320000
_D = 128
_H = 8
_DH = 16
_FC = 64

_NC = 2
_NS = 16
_NW = _NC * _NS
_EPT = _E // _NW
_B = 40
_NCH = _EPT // _B
_GOFF = (0, 16, 24)
_GRP = _B // 16
_AW = 128
_DBASE = 10000
_ANR = 10752
_RPT = _ANR // _NS
_ZC = 8

_MESH = plsc.VectorSubcoreMesh(
    core_axis_name="c", subcore_axis_name="s", num_cores=_NC, num_subcores=_NS)

_f32 = jnp.float32
_i32 = jnp.int32


def _len_body(px, py, pz, src, dst, out,
              sib, dib, sxb, syb, szb, dxb, dyb, dzb, lb, sem):
  c = lax.axis_index("c")
  s = lax.axis_index("s")
  wid = c * _NS + s

  def chunk(ci, carry):
    base = wid * _EPT + ci * _B
    pltpu.sync_copy(src.at[pl.ds(base, _B)], sib)
    pltpu.sync_copy(dst.at[pl.ds(base, _B)], dib)
    d1 = pltpu.async_copy(px.at[sib], sxb, sem)
    d2 = pltpu.async_copy(py.at[sib], syb, sem)
    d3 = pltpu.async_copy(pz.at[sib], szb, sem)
    d4 = pltpu.async_copy(px.at[dib], dxb, sem)
    d5 = pltpu.async_copy(py.at[dib], dyb, sem)
    d6 = pltpu.async_copy(pz.at[dib], dzb, sem)
    d1.wait(); d2.wait(); d3.wait(); d4.wait(); d5.wait(); d6.wait()

    def grp(g, carry2):
      sl = pl.ds(g * 16, 16)
      dx = sxb[sl] - dxb[sl]
      dy = syb[sl] - dyb[sl]
      dz = szb[sl] - dzb[sl]
      lb[sl] = dx * dx + dy * dy + dz * dz + 1e-12
      return carry2

    lax.fori_loop(0, _GRP, grp, 0)
    pltpu.sync_copy(lb, out.at[pl.ds(base, _B)])
    return carry

  lax.fori_loop(0, _NCH, chunk, 0)


def _edge_len_sc(px, py, pz, src, dst):
  k = pl.kernel(
      _len_body,
      out_type=jax.ShapeDtypeStruct((_E,), _f32),
      mesh=_MESH,
      compiler_params=pltpu.CompilerParams(needs_layout_passes=False),
      scratch_types=[
          pltpu.VMEM((_B,), _i32), pltpu.VMEM((_B,), _i32),
          pltpu.VMEM((_B,), _f32), pltpu.VMEM((_B,), _f32),
          pltpu.VMEM((_B,), _f32), pltpu.VMEM((_B,), _f32),
          pltpu.VMEM((_B,), _f32), pltpu.VMEM((_B,), _f32),
          pltpu.VMEM((_B,), _f32),
          pltpu.SemaphoreType.DMA,
      ],
  )
  return k(px, py, pz, src, dst)


def _att_body(qh, kh, vh, srch, dsth, ebh, outh,
              sI, dI, dwib, qd, ks2, vs2, rows, wrows, ebb, zc, acc, semG,
              semQ):
  c = lax.axis_index("c")
  s = lax.axis_index("s")
  wid = c * _NS + s
  zero16 = jnp.zeros((16,), _f32)

  def zrow(i, carry):
    for j in range(_AW // 16):
      zc[i, pl.ds(j * 16, 16)] = zero16
    return carry

  lax.fori_loop(0, _ZC, zrow, 0)
  r0 = s * _RPT

  def zacc(j, carry):
    pltpu.sync_copy(zc, acc.at[pl.ds(r0 + j * _ZC, _ZC)])
    return carry

  lax.fori_loop(0, _RPT // _ZC, zacc, 0)

  def zrow2(i, carry):
    for j in range(_AW // 16):
      wrows[i, pl.ds(j * 16, 16)] = zero16
    return carry

  lax.fori_loop(0, _B, zrow2, 0)
  plsc.subcore_barrier()

  def chunk(ci, carry):
    pn = lax.rem(ci + 1, 2)

    @pl.when(ci + 1 < _NCH)
    def _():
      base_n = wid * _EPT + (ci + 1) * _B
      pltpu.sync_copy(srch.at[pl.ds(base_n, _B)], sI.at[pn])
      pltpu.sync_copy(dsth.at[pl.ds(base_n, _B)], dI.at[pn])
      pltpu.async_copy(kh.at[sI.at[pn]], ks2.at[pn], semG.at[pn])
      pltpu.async_copy(vh.at[sI.at[pn]], vs2.at[pn], semG.at[pn])

    @pl.when(ci >= 0)
    def _():
      p = lax.rem(ci, 2)
      base = wid * _EPT + ci * _B
      pltpu.sync_copy(ebh.at[pl.ds(base, _B)], ebb)
      dq = pltpu.async_copy(qh.at[dI.at[p]], qd, semQ)
      pltpu.make_async_copy(kh.at[pl.ds(0, _B)], ks2.at[p], semG.at[p]).wait()
      pltpu.make_async_copy(vh.at[pl.ds(0, _B)], vs2.at[p], semG.at[p]).wait()
      dq.wait()
      pv = jnp.full((16,), p, _i32)

      for goff in _GOFF:
        eidx = goff + lax.iota(_i32, 16)
        dstv = dI[p, pl.ds(goff, 16)]
        dwib[pl.ds(goff, 16)] = _DBASE + (dstv >> 4)
        dcol = (dstv & 15) * _H
        for h in range(_H):
          a = jnp.zeros((16,), _f32)
          for t in range(_DH):
            d = h * _DH + t
            cd = jnp.full((16,), d, _i32)
            a = a + (plsc.load_gather(qd, [eidx, cd])
                     * plsc.load_gather(ks2, [pv, eidx, cd]))
          w = jnp.exp(a) * plsc.load_gather(ebb, [eidx, jnp.full((16,), h, _i32)])
          for t in range(_DH):
            d = h * _DH + t
            cd = jnp.full((16,), d, _i32)
            vv = plsc.load_gather(vs2, [pv, eidx, cd])
            plsc.store_scatter(rows, [eidx, cd], vv * w)
          plsc.store_scatter(wrows, [eidx, dcol + h], w)

      pltpu.sync_copy(rows, acc.at[dI.at[p]], add=True)
      pltpu.sync_copy(wrows, acc.at[dwib], add=True)

      for goff in _GOFF:
        eidx = goff + lax.iota(_i32, 16)
        dcol = (dI[p, pl.ds(goff, 16)] & 15) * _H
        for h in range(_H):
          plsc.store_scatter(wrows, [eidx, dcol + h], zero16)
    return carry

  lax.fori_loop(-1, _NCH, chunk, 0)
  plsc.subcore_barrier()

  def dump(j, carry):
    pltpu.sync_copy(acc.at[pl.ds(r0 + j * _ZC, _ZC)], zc)
    pltpu.sync_copy(zc, outh.at[pl.ds(c * _ANR + r0 + j * _ZC, _ZC)])
    return carry

  lax.fori_loop(0, _RPT // _ZC, dump, 0)


def _edge_pass_sc(q, k, v, src, dst, ebt):
  kk = pl.kernel(
      _att_body,
      out_type=jax.ShapeDtypeStruct((_NC * _ANR, _AW), _f32),
      mesh=_MESH,
      compiler_params=pltpu.CompilerParams(needs_layout_passes=False),
      scratch_types=[
          pltpu.VMEM((2, _B), _i32), pltpu.VMEM((2, _B), _i32),
          pltpu.VMEM((_B,), _i32),
          pltpu.VMEM((_B, _D), _f32), pltpu.VMEM((2, _B, _D), _f32),
          pltpu.VMEM((2, _B, _D), _f32),
          pltpu.VMEM((_B, _AW), _f32),
          pltpu.VMEM((_B, _AW), _f32),
          pltpu.VMEM((_B, _H), _f32),
          pltpu.VMEM((_ZC, _AW), _f32),
          pltpu.VMEM_SHARED((_ANR, _AW), _f32),
          pltpu.SemaphoreType.DMA((2,)),
          pltpu.SemaphoreType.DMA,
      ],
  )
  return kk(q, k, v, src, dst, ebt)


def _qkv_tc(h, attr, wq, wk, wv):
  T = 1000

  def body(h_ref, a_ref, wq_ref, wk_ref, wv_ref, q_ref, k_ref, v_ref):
    hg = h_ref[...] * a_ref[...]
    q_ref[...] = jnp.dot(hg, wq_ref[...], preferred_element_type=_f32) * (1.0 / 4.0)
    k_ref[...] = jnp.dot(hg, wk_ref[...], preferred_element_type=_f32)
    v_ref[...] = jnp.dot(hg, wv_ref[...], preferred_element_type=_f32)

  w_spec = pl.BlockSpec((_D, _D), lambda i: (0, 0))
  return pl.pallas_call(
      body,
      grid=(_N // T,),
      in_specs=[pl.BlockSpec((T, _D), lambda i: (i, 0)),
                pl.BlockSpec((T, 1), lambda i: (i, 0)),
                w_spec, w_spec, w_spec],
      out_specs=[pl.BlockSpec((T, _D), lambda i: (i, 0))] * 3,
      out_shape=[jax.ShapeDtypeStruct((_N, _D), _f32)] * 3,
  )(h, attr, wq, wk, wv)


def _qkv3_tc(oi, of, p2, attr, wq, wk, wv):
  T = 1000

  def body(oi_ref, of_ref, p_ref, a_ref, wq_ref, wk_ref, wv_ref,
           q_ref, k_ref, v_ref):
    ps = p_ref[0, 0]
    hg = ((1.0 - ps) * oi_ref[...] + ps * of_ref[...]) * a_ref[...]
    q_ref[...] = jnp.dot(hg, wq_ref[...], preferred_element_type=_f32) * (1.0 / 4.0)
    k_ref[...] = jnp.dot(hg, wk_ref[...], preferred_element_type=_f32)
    v_ref[...] = jnp.dot(hg, wv_ref[...], preferred_element_type=_f32)

  w_spec = pl.BlockSpec((_D, _D), lambda i: (0, 0))
  return pl.pallas_call(
      body,
      grid=(_N // T,),
      in_specs=[pl.BlockSpec((T, _D), lambda i: (i, 0)),
                pl.BlockSpec((T, _D), lambda i: (i, 0)),
                pl.BlockSpec((1, 1), lambda i: (0, 0)),
                pl.BlockSpec((T, 1), lambda i: (i, 0)),
                w_spec, w_spec, w_spec],
      out_specs=[pl.BlockSpec((T, _D), lambda i: (i, 0))] * 3,
      out_shape=[jax.ShapeDtypeStruct((_N, _D), _f32)] * 3,
  )(oi, of, p2, attr, wq, wk, wv)


def _bias_tc(len2, ew1, eb1, ew2):
  T = 3200

  def body(l_ref, w1_ref, b1_ref, w2_ref, o_ref):
    ln = jnp.sqrt(l_ref[...])
    feat = jnp.tanh(ln * w1_ref[...] + b1_ref[...])
    o_ref[...] = jnp.exp(
        jnp.dot(feat, w2_ref[...], preferred_element_type=_f32))

  return pl.pallas_call(
      body,
      grid=(_E // T,),
      in_specs=[pl.BlockSpec((T, 1), lambda i: (i, 0)),
                pl.BlockSpec((1, _FC), lambda i: (0, 0)),
                pl.BlockSpec((1, _FC), lambda i: (0, 0)),
                pl.BlockSpec((_FC, _H), lambda i: (0, 0))],
      out_specs=pl.BlockSpec((T, _H), lambda i: (i, 0)),
      out_shape=jax.ShapeDtypeStruct((_E, _H), _f32),
  )(len2, ew1, eb1, ew2)


def _finish_tc(parts, wo, expand):
  T = 1000
  parts3 = parts.reshape(_NC, _ANR, _AW)
  den_a = parts3[0, _DBASE:_DBASE + _N // 16, :].reshape(-1, _H)
  den_b = parts3[1, _DBASE:_DBASE + _N // 16, :].reshape(-1, _H)

  def body(p_ref, da_ref, db_ref, wo_ref, ex_ref, o_ref):
    num = p_ref[0] + p_ref[1]
    den = da_ref[...] + db_ref[...]
    rec = 1.0 / (den + 1e-9)
    rec128 = jnp.dot(rec, ex_ref[...], preferred_element_type=_f32)
    o_ref[...] = jnp.dot(num * rec128, wo_ref[...], preferred_element_type=_f32)

  return pl.pallas_call(
      body,
      grid=(_N // T,),
      in_specs=[pl.BlockSpec((_NC, T, _AW), lambda i: (0, i, 0)),
                pl.BlockSpec((T, _H), lambda i: (i, 0)),
                pl.BlockSpec((T, _H), lambda i: (i, 0)),
                pl.BlockSpec((_D, _D), lambda i: (0, 0)),
                pl.BlockSpec((_H, _D), lambda i: (0, 0))],
      out_specs=pl.BlockSpec((T, _D), lambda i: (i, 0)),
      out_shape=jax.ShapeDtypeStruct((_N, _D), _f32),
  )(parts3, den_a, den_b, wo, expand)


def _block_pre(pos, src, dst, prm):
  ln = _edge_len_sc(pos[:, 0], pos[:, 1], pos[:, 2], src, dst)
  return _bias_tc(ln.reshape(_E, 1), prm["ew1"], prm["eb1"].reshape(1, _FC),
                  prm["ew2"])


def kernel(x, node_attr, pos, edge_index, x_final_state, pos_final_state,
           edge_index_final_state, pos_interpolated_transition_state,
           edge_index_interpolated_transition_state, batch, p,
           params_init, params_final, params_ts):
  del batch
  expand = jnp.asarray(np.kron(np.eye(_H), np.ones((1, _DH))), dtype=_f32)

  src1, dst1 = edge_index[0], edge_index[1]
  src2, dst2 = edge_index_final_state[0], edge_index_final_state[1]
  src3, dst3 = (edge_index_interpolated_transition_state[0],
                edge_index_interpolated_transition_state[1])

  eb1 = _block_pre(pos, src1, dst1, params_init)
  eb2 = _block_pre(pos_final_state, src2, dst2, params_final)
  eb3 = _block_pre(pos_interpolated_transition_state, src3, dst3, params_ts)

  q1, k1, v1 = _qkv_tc(x, node_attr, params_init["Wq"], params_init["Wk"],
                       params_init["Wv"])
  parts1 = _edge_pass_sc(q1, k1, v1, src1, dst1, eb1)
  out1 = _finish_tc(parts1, params_init["Wo"], expand)

  q2, k2, v2 = _qkv_tc(x_final_state, node_attr, params_final["Wq"],
                       params_final["Wk"], params_final["Wv"])
  parts2 = _edge_pass_sc(q2, k2, v2, src2, dst2, eb2)
  out2 = _finish_tc(parts2, params_final["Wo"], expand)

  q3, k3, v3 = _qkv3_tc(out1, out2, p.reshape(1, 1), node_attr,
                        params_ts["Wq"], params_ts["Wk"], params_ts["Wv"])
  parts3 = _edge_pass_sc(q3, k3, v3, src3, dst3, eb3)
  return _finish_tc(parts3, params_ts["Wo"], expand)


# speedup vs baseline: 1.7533x; 1.7533x over previous
import numpy as np
import jax
import jax.numpy as jnp
from jax import lax
from jax.experimental import pallas as pl
from jax.experimental.pallas import tpu as pltpu
from jax.experimental.pallas import tpu_sc as plsc

_N = 10000
_E = 320000
_D = 128
_H = 8
_DH = 16
_FC = 64

_NC = 2
_NS = 16
_NW = _NC * _NS
_EPT = _E // _NW
_B = 40
_NCH = _EPT // _B
_GOFF = (0, 16, 24)
_GRP = _B // 16
_AW = 128
_NPAD = 10240
_ANR = 11264
_RPT = _ANR // _NS
_ZC = 64

_MESH = plsc.VectorSubcoreMesh(
    core_axis_name="c", subcore_axis_name="s", num_cores=_NC, num_subcores=_NS)

_f32 = jnp.float32
_i32 = jnp.int32


def _len_body(px, py, pz, src, dst, out,
              sib, dib, sxb, syb, szb, dxb, dyb, dzb, lb, sem):
  c = lax.axis_index("c")
  s = lax.axis_index("s")
  wid = c * _NS + s

  def chunk(ci, carry):
    base = wid * _EPT + ci * _B
    pltpu.sync_copy(src.at[pl.ds(base, _B)], sib)
    pltpu.sync_copy(dst.at[pl.ds(base, _B)], dib)
    d1 = pltpu.async_copy(px.at[sib], sxb, sem)
    d2 = pltpu.async_copy(py.at[sib], syb, sem)
    d3 = pltpu.async_copy(pz.at[sib], szb, sem)
    d4 = pltpu.async_copy(px.at[dib], dxb, sem)
    d5 = pltpu.async_copy(py.at[dib], dyb, sem)
    d6 = pltpu.async_copy(pz.at[dib], dzb, sem)
    d1.wait(); d2.wait(); d3.wait(); d4.wait(); d5.wait(); d6.wait()

    def grp(g, carry2):
      sl = pl.ds(g * 16, 16)
      dx = sxb[sl] - dxb[sl]
      dy = syb[sl] - dyb[sl]
      dz = szb[sl] - dzb[sl]
      lb[sl] = dx * dx + dy * dy + dz * dz + 1e-12
      return carry2

    lax.fori_loop(0, _GRP, grp, 0)
    pltpu.sync_copy(lb, out.at[pl.ds(base, _B)])
    return carry

  lax.fori_loop(0, _NCH, chunk, 0)


def _edge_len_sc(px, py, pz, src, dst):
  k = pl.kernel(
      _len_body,
      out_type=jax.ShapeDtypeStruct((_E,), _f32),
      mesh=_MESH,
      compiler_params=pltpu.CompilerParams(needs_layout_passes=False),
      scratch_types=[
          pltpu.VMEM((_B,), _i32), pltpu.VMEM((_B,), _i32),
          pltpu.VMEM((_B,), _f32), pltpu.VMEM((_B,), _f32),
          pltpu.VMEM((_B,), _f32), pltpu.VMEM((_B,), _f32),
          pltpu.VMEM((_B,), _f32), pltpu.VMEM((_B,), _f32),
          pltpu.VMEM((_B,), _f32),
          pltpu.SemaphoreType.DMA,
      ],
  )
  return k(px, py, pz, src, dst)


def _att_body(qh, kh, vh, srch, dsth, ebh, outh,
              sib, dib, dwib, dcolb, qd, ks, vs, rows, wrows, ebb, tmpl, zc,
              acc, sem):
  c = lax.axis_index("c")
  s = lax.axis_index("s")
  wid = c * _NS + s
  zero16 = jnp.zeros((16,), _f32)
  iota16 = lax.iota(_i32, 16)
  m15 = iota16 == 15
  m8 = iota16 < _H

  def zrow(i, carry):
    for j in range(_AW // 16):
      zc[i, pl.ds(j * 16, 16)] = zero16
    return carry

  lax.fori_loop(0, _ZC, zrow, 0)
  r0 = s * _RPT
  for j in range(_RPT // _ZC):
    pltpu.sync_copy(zc, acc.at[pl.ds(r0 + j * _ZC, _ZC)])

  def zrow2(i, carry):
    for j in range(_AW // 16):
      wrows[i, pl.ds(j * 16, 16)] = zero16
    return carry

  lax.fori_loop(0, _B, zrow2, 0)
  plsc.subcore_barrier()

  def chunk(ci, carry):
    base = wid * _EPT + ci * _B
    pltpu.sync_copy(srch.at[pl.ds(base, _B)], sib)
    pltpu.sync_copy(dsth.at[pl.ds(base, _B)], dib)
    dq = pltpu.async_copy(qh.at[dib], qd, sem)
    dk = pltpu.async_copy(kh.at[sib], ks, sem)
    dv = pltpu.async_copy(vh.at[sib], vs, sem)
    pltpu.sync_copy(ebh.at[pl.ds(base, _B), :], ebb)
    dq.wait(); dk.wait(); dv.wait()

    for goff in _GOFF:
      dstv = dib[pl.ds(goff, 16)]
      dwib[pl.ds(goff, 16)] = _NPAD + (dstv >> 4)
      dcolb[pl.ds(goff, 16)] = (dstv & 15) * _H

    def edge(e, carry2):
      dcol = dcolb[pl.ds(e, 16)][0]
      for h in range(_H):
        prod = qd[e, pl.ds(h * _DH, _DH)] * ks[e, pl.ds(h * _DH, _DH)]
        cs = plsc.cumsum(prod)
        plsc.store_scatter(tmpl, [jnp.full((16,), h, _i32)], cs, mask=m15)
      w16 = jnp.exp(tmpl[pl.ds(0, 16)] + ebb[e, pl.ds(0, 16)])
      plsc.store_scatter(wrows, [jnp.full((16,), e, _i32), dcol + iota16],
                         w16, mask=m8)
      for h in range(_H):
        rows[e, pl.ds(h * _DH, _DH)] = vs[e, pl.ds(h * _DH, _DH)] * w16[h]
      return carry2

    lax.fori_loop(0, _B, edge, 0)
    pltpu.sync_copy(rows, acc.at[dib], add=True)
    pltpu.sync_copy(wrows, acc.at[dwib], add=True)

    def clr(e, carry2):
      dcol = dcolb[pl.ds(e, 16)][0]
      plsc.store_scatter(wrows, [jnp.full((16,), e, _i32), dcol + iota16],
                         zero16, mask=m8)
      return carry2

    lax.fori_loop(0, _B, clr, 0)
    return carry

  lax.fori_loop(0, _NCH, chunk, 0)
  plsc.subcore_barrier()

  for j in range(_RPT // _ZC):
    pltpu.sync_copy(acc.at[pl.ds(r0 + j * _ZC, _ZC)], zc)
    pltpu.sync_copy(zc, outh.at[pl.ds(c * _ANR + r0 + j * _ZC, _ZC)])


def _edge_pass_sc(q, k, v, src, dst, ebt):
  kk = pl.kernel(
      _att_body,
      out_type=jax.ShapeDtypeStruct((_NC * _ANR, _AW), _f32),
      mesh=_MESH,
      compiler_params=pltpu.CompilerParams(needs_layout_passes=False),
      scratch_types=[
          pltpu.VMEM((_B,), _i32), pltpu.VMEM((_B,), _i32),
          pltpu.VMEM((_B,), _i32), pltpu.VMEM((_B + 16,), _i32),
          pltpu.VMEM((_B, _D), _f32), pltpu.VMEM((_B, _D), _f32),
          pltpu.VMEM((_B, _D), _f32),
          pltpu.VMEM((_B, _AW), _f32),
          pltpu.VMEM((_B, _AW), _f32),
          pltpu.VMEM((_B, 16), _f32),
          pltpu.VMEM((16,), _f32),
          pltpu.VMEM((_ZC, _AW), _f32),
          pltpu.VMEM_SHARED((_ANR, _AW), _f32),
          pltpu.SemaphoreType.DMA,
      ],
  )
  return kk(q, k, v, src, dst, ebt)


def _qkv_tc(h, attr, wq, wk, wv):
  T = 1000

  def body(h_ref, a_ref, wq_ref, wk_ref, wv_ref, q_ref, k_ref, v_ref):
    hg = h_ref[...] * a_ref[...]
    q_ref[...] = jnp.dot(hg, wq_ref[...], preferred_element_type=_f32) * (1.0 / 4.0)
    k_ref[...] = jnp.dot(hg, wk_ref[...], preferred_element_type=_f32)
    v_ref[...] = jnp.dot(hg, wv_ref[...], preferred_element_type=_f32)

  w_spec = pl.BlockSpec((_D, _D), lambda i: (0, 0))
  return pl.pallas_call(
      body,
      grid=(_N // T,),
      in_specs=[pl.BlockSpec((T, _D), lambda i: (i, 0)),
                pl.BlockSpec((T, 1), lambda i: (i, 0)),
                w_spec, w_spec, w_spec],
      out_specs=[pl.BlockSpec((T, _D), lambda i: (i, 0))] * 3,
      out_shape=[jax.ShapeDtypeStruct((_N, _D), _f32)] * 3,
  )(h, attr, wq, wk, wv)


def _qkv3_tc(oi, of, p2, attr, wq, wk, wv):
  T = 1000

  def body(oi_ref, of_ref, p_ref, a_ref, wq_ref, wk_ref, wv_ref,
           q_ref, k_ref, v_ref):
    ps = p_ref[0, 0]
    hg = ((1.0 - ps) * oi_ref[...] + ps * of_ref[...]) * a_ref[...]
    q_ref[...] = jnp.dot(hg, wq_ref[...], preferred_element_type=_f32) * (1.0 / 4.0)
    k_ref[...] = jnp.dot(hg, wk_ref[...], preferred_element_type=_f32)
    v_ref[...] = jnp.dot(hg, wv_ref[...], preferred_element_type=_f32)

  w_spec = pl.BlockSpec((_D, _D), lambda i: (0, 0))
  return pl.pallas_call(
      body,
      grid=(_N // T,),
      in_specs=[pl.BlockSpec((T, _D), lambda i: (i, 0)),
                pl.BlockSpec((T, _D), lambda i: (i, 0)),
                pl.BlockSpec((1, 1), lambda i: (0, 0)),
                pl.BlockSpec((T, 1), lambda i: (i, 0)),
                w_spec, w_spec, w_spec],
      out_specs=[pl.BlockSpec((T, _D), lambda i: (i, 0))] * 3,
      out_shape=[jax.ShapeDtypeStruct((_N, _D), _f32)] * 3,
  )(oi, of, p2, attr, wq, wk, wv)


def _bias_tc(len2, ew1, eb1, ew2):
  T = 3200

  def body(l_ref, w1_ref, b1_ref, w2_ref, o_ref):
    ln = jnp.sqrt(l_ref[...])
    feat = jnp.tanh(ln * w1_ref[...] + b1_ref[...])
    o_ref[...] = jnp.dot(feat, w2_ref[...], preferred_element_type=_f32)

  return pl.pallas_call(
      body,
      grid=(_E // T,),
      in_specs=[pl.BlockSpec((T, 1), lambda i: (i, 0)),
                pl.BlockSpec((1, _FC), lambda i: (0, 0)),
                pl.BlockSpec((1, _FC), lambda i: (0, 0)),
                pl.BlockSpec((_FC, 16), lambda i: (0, 0))],
      out_specs=pl.BlockSpec((T, 16), lambda i: (i, 0)),
      out_shape=jax.ShapeDtypeStruct((_E, 16), _f32),
  )(len2, ew1, eb1, jnp.pad(ew2, ((0, 0), (0, 8))))


def _finish_tc(parts, wo, expand):
  T = 1000
  parts3 = parts.reshape(_NC, _ANR, _AW)
  den_a = parts3[0, _NPAD:_NPAD + _N // 16, :].reshape(-1, _H)
  den_b = parts3[1, _NPAD:_NPAD + _N // 16, :].reshape(-1, _H)

  def body(p_ref, da_ref, db_ref, wo_ref, ex_ref, o_ref):
    num = p_ref[0] + p_ref[1]
    den = da_ref[...] + db_ref[...]
    rec = 1.0 / (den + 1e-9)
    rec128 = jnp.dot(rec, ex_ref[...], preferred_element_type=_f32)
    o_ref[...] = jnp.dot(num * rec128, wo_ref[...], preferred_element_type=_f32)

  return pl.pallas_call(
      body,
      grid=(_N // T,),
      in_specs=[pl.BlockSpec((_NC, T, _AW), lambda i: (0, i, 0)),
                pl.BlockSpec((T, _H), lambda i: (i, 0)),
                pl.BlockSpec((T, _H), lambda i: (i, 0)),
                pl.BlockSpec((_D, _D), lambda i: (0, 0)),
                pl.BlockSpec((_H, _D), lambda i: (0, 0))],
      out_specs=pl.BlockSpec((T, _D), lambda i: (i, 0)),
      out_shape=jax.ShapeDtypeStruct((_N, _D), _f32),
  )(parts3, den_a, den_b, wo, expand)


def _block_pre(pos, src, dst, prm):
  ln = _edge_len_sc(pos[:, 0], pos[:, 1], pos[:, 2], src, dst)
  return _bias_tc(ln.reshape(_E, 1), prm["ew1"], prm["eb1"].reshape(1, _FC),
                  prm["ew2"])


def kernel(x, node_attr, pos, edge_index, x_final_state, pos_final_state,
           edge_index_final_state, pos_interpolated_transition_state,
           edge_index_interpolated_transition_state, batch, p,
           params_init, params_final, params_ts):
  del batch
  expand = jnp.asarray(np.kron(np.eye(_H), np.ones((1, _DH))), dtype=_f32)

  src1, dst1 = edge_index[0], edge_index[1]
  src2, dst2 = edge_index_final_state[0], edge_index_final_state[1]
  src3, dst3 = (edge_index_interpolated_transition_state[0],
                edge_index_interpolated_transition_state[1])

  eb1 = _block_pre(pos, src1, dst1, params_init)
  eb2 = _block_pre(pos_final_state, src2, dst2, params_final)
  eb3 = _block_pre(pos_interpolated_transition_state, src3, dst3, params_ts)

  q1, k1, v1 = _qkv_tc(x, node_attr, params_init["Wq"], params_init["Wk"],
                       params_init["Wv"])
  parts1 = _edge_pass_sc(q1, k1, v1, src1, dst1, eb1)
  out1 = _finish_tc(parts1, params_init["Wo"], expand)

  q2, k2, v2 = _qkv_tc(x_final_state, node_attr, params_final["Wq"],
                       params_final["Wk"], params_final["Wv"])
  parts2 = _edge_pass_sc(q2, k2, v2, src2, dst2, eb2)
  out2 = _finish_tc(parts2, params_final["Wo"], expand)

  q3, k3, v3 = _qkv3_tc(out1, out2, p.reshape(1, 1), node_attr,
                        params_ts["Wq"], params_ts["Wk"], params_ts["Wv"])
  parts3 = _edge_pass_sc(q3, k3, v3, src3, dst3, eb3)
  return _finish_tc(parts3, params_ts["Wo"], expand)


# speedup vs baseline: 1.9829x; 1.1310x over previous
import numpy as np
import jax
import jax.numpy as jnp
from jax import lax
from jax.experimental import pallas as pl
from jax.experimental.pallas import tpu as pltpu
from jax.experimental.pallas import tpu_sc as plsc

_N = 10000
_E = 320000
_D = 128
_H = 8
_DH = 16
_FC = 64

_NC = 2
_NS = 16
_NW = _NC * _NS
_EPT = _E // _NW
_B = 40
_NCH = _EPT // _B
_GOFF = (0, 16, 24)
_GRP = _B // 16
_AW = 128
_NPAD = 10240
_ANR = 11264
_RPT = _ANR // _NS
_ZC = 64

_MESH = plsc.VectorSubcoreMesh(
    core_axis_name="c", subcore_axis_name="s", num_cores=_NC, num_subcores=_NS)

_f32 = jnp.float32
_i32 = jnp.int32


_LB = 400


def _len_body(px, py, pz, src, dst, out,
              sib, dib, sxb, syb, szb, dxb, dyb, dzb, lb, sem):
  c = lax.axis_index("c")
  s = lax.axis_index("s")
  wid = c * _NS + s

  def chunk(ci, carry):
    base = wid * _EPT + ci * _LB
    pltpu.sync_copy(src.at[pl.ds(base, _LB)], sib)
    pltpu.sync_copy(dst.at[pl.ds(base, _LB)], dib)
    d1 = pltpu.async_copy(px.at[sib], sxb, sem)
    d2 = pltpu.async_copy(py.at[sib], syb, sem)
    d3 = pltpu.async_copy(pz.at[sib], szb, sem)
    d4 = pltpu.async_copy(px.at[dib], dxb, sem)
    d5 = pltpu.async_copy(py.at[dib], dyb, sem)
    d6 = pltpu.async_copy(pz.at[dib], dzb, sem)
    d1.wait(); d2.wait(); d3.wait(); d4.wait(); d5.wait(); d6.wait()

    def grp(g, carry2):
      sl = pl.ds(g * 16, 16)
      dx = sxb[sl] - dxb[sl]
      dy = syb[sl] - dyb[sl]
      dz = szb[sl] - dzb[sl]
      lb[sl] = dx * dx + dy * dy + dz * dz + 1e-12
      return carry2

    lax.fori_loop(0, _LB // 16, grp, 0)
    pltpu.sync_copy(lb, out.at[pl.ds(base, _LB)])
    return carry

  lax.fori_loop(0, _EPT // _LB, chunk, 0)


def _edge_len_sc(px, py, pz, src, dst):
  k = pl.kernel(
      _len_body,
      out_type=jax.ShapeDtypeStruct((_E,), _f32),
      mesh=_MESH,
      compiler_params=pltpu.CompilerParams(needs_layout_passes=False),
      scratch_types=[
          pltpu.VMEM((_LB,), _i32), pltpu.VMEM((_LB,), _i32),
          pltpu.VMEM((_LB,), _f32), pltpu.VMEM((_LB,), _f32),
          pltpu.VMEM((_LB,), _f32), pltpu.VMEM((_LB,), _f32),
          pltpu.VMEM((_LB,), _f32), pltpu.VMEM((_LB,), _f32),
          pltpu.VMEM((_LB,), _f32),
          pltpu.SemaphoreType.DMA,
      ],
  )
  return k(px, py, pz, src, dst)


def _att_body(qh, kh, vh, srch, dsth, ebh, outh,
              sib, dib, dwib, dcolb, qd, ks, vs, rows, wrows, ebb,
              tmpl, zc, acc, sem):
  c = lax.axis_index("c")
  s = lax.axis_index("s")
  wid = c * _NS + s
  zero16 = jnp.zeros((16,), _f32)
  iota16 = lax.iota(_i32, 16)
  m15 = iota16 == 15
  m8 = iota16 < _H

  def zrow(i, carry):
    for j in range(_AW // 16):
      zc[i, pl.ds(j * 16, 16)] = zero16
    return carry

  lax.fori_loop(0, _ZC, zrow, 0)
  r0 = s * _RPT
  for j in range(_RPT // _ZC):
    pltpu.sync_copy(zc, acc.at[pl.ds(r0 + j * _ZC, _ZC)])

  def zrow2(i, carry):
    for j in range(_AW // 16):
      wrows[i, pl.ds(j * 16, 16)] = zero16
    return carry

  lax.fori_loop(0, _B, zrow2, 0)
  plsc.subcore_barrier()

  def chunk(ci, carry):
    base = wid * _EPT + ci * _B
    pltpu.sync_copy(srch.at[pl.ds(base, _B)], sib)
    pltpu.sync_copy(dsth.at[pl.ds(base, _B)], dib)
    dq = pltpu.async_copy(qh.at[dib], qd, sem)
    dk = pltpu.async_copy(kh.at[sib], ks, sem)
    dv = pltpu.async_copy(vh.at[sib], vs, sem)
    pltpu.sync_copy(ebh.at[pl.ds(base, _B), :], ebb)

    for goff in _GOFF:
      dstv = dib[pl.ds(goff, 16)]
      dwib[pl.ds(goff, 16)] = _NPAD + (dstv >> 4)
      dcolb[pl.ds(goff, 16)] = (dstv & 15) * _H
    dq.wait(); dk.wait(); dv.wait()

    def edge(e, carry2):
      dcol = dcolb[pl.ds(e, 16)][0]
      for h in range(_H):
        prod = qd[e, pl.ds(h * _DH, _DH)] * ks[e, pl.ds(h * _DH, _DH)]
        cs = plsc.cumsum(prod)
        plsc.store_scatter(tmpl, [jnp.full((16,), h, _i32)], cs, mask=m15)
      w16 = jnp.exp(tmpl[pl.ds(0, 16)] + ebb[e, pl.ds(0, 16)])
      plsc.store_scatter(wrows, [jnp.full((16,), e, _i32), dcol + iota16],
                         w16, mask=m8)
      for h in range(_H):
        rows[e, pl.ds(h * _DH, _DH)] = vs[e, pl.ds(h * _DH, _DH)] * w16[h]
      return carry2

    lax.fori_loop(0, _B, edge, 0)
    pltpu.sync_copy(rows, acc.at[dib], add=True)
    pltpu.sync_copy(wrows, acc.at[dwib], add=True)

    def clr(e, carry2):
      dcol = dcolb[pl.ds(e, 16)][0]
      plsc.store_scatter(wrows, [jnp.full((16,), e, _i32), dcol + iota16],
                         zero16, mask=m8)
      return carry2

    lax.fori_loop(0, _B, clr, 0)
    return carry

  lax.fori_loop(0, _NCH, chunk, 0)
  plsc.subcore_barrier()

  for j in range(_RPT // _ZC):
    pltpu.sync_copy(acc.at[pl.ds(r0 + j * _ZC, _ZC)], zc)
    pltpu.sync_copy(zc, outh.at[pl.ds(c * _ANR + r0 + j * _ZC, _ZC)])


def _edge_pass_sc(q, k, v, src, dst, ebt):
  kk = pl.kernel(
      _att_body,
      out_type=jax.ShapeDtypeStruct((_NC * _ANR, _AW), _f32),
      mesh=_MESH,
      compiler_params=pltpu.CompilerParams(needs_layout_passes=False),
      scratch_types=[
          pltpu.VMEM((_B,), _i32), pltpu.VMEM((_B,), _i32),
          pltpu.VMEM((_B,), _i32),
          pltpu.VMEM((_B + 16,), _i32),
          pltpu.VMEM((_B, _D), _f32), pltpu.VMEM((_B, _D), _f32),
          pltpu.VMEM((_B, _D), _f32),
          pltpu.VMEM((_B, _AW), _f32),
          pltpu.VMEM((_B, _AW), _f32),
          pltpu.VMEM((_B, 16), _f32),
          pltpu.VMEM((16,), _f32),
          pltpu.VMEM((_ZC, _AW), _f32),
          pltpu.VMEM_SHARED((_ANR, _AW), _f32),
          pltpu.SemaphoreType.DMA,
      ],
  )
  return kk(q, k, v, src, dst, ebt)


def _qkv_tc(h, attr, wq, wk, wv):
  T = 1000

  def body(h_ref, a_ref, wq_ref, wk_ref, wv_ref, q_ref, k_ref, v_ref):
    hg = h_ref[...] * a_ref[...]
    q_ref[...] = jnp.dot(hg, wq_ref[...], preferred_element_type=_f32) * (1.0 / 4.0)
    k_ref[...] = jnp.dot(hg, wk_ref[...], preferred_element_type=_f32)
    v_ref[...] = jnp.dot(hg, wv_ref[...], preferred_element_type=_f32)

  w_spec = pl.BlockSpec((_D, _D), lambda i: (0, 0))
  return pl.pallas_call(
      body,
      grid=(_N // T,),
      in_specs=[pl.BlockSpec((T, _D), lambda i: (i, 0)),
                pl.BlockSpec((T, 1), lambda i: (i, 0)),
                w_spec, w_spec, w_spec],
      out_specs=[pl.BlockSpec((T, _D), lambda i: (i, 0))] * 3,
      out_shape=[jax.ShapeDtypeStruct((_N, _D), _f32)] * 3,
  )(h, attr, wq, wk, wv)


def _qkv3_tc(oi, of, p2, attr, wq, wk, wv):
  T = 1000

  def body(oi_ref, of_ref, p_ref, a_ref, wq_ref, wk_ref, wv_ref,
           q_ref, k_ref, v_ref):
    ps = p_ref[0, 0]
    hg = ((1.0 - ps) * oi_ref[...] + ps * of_ref[...]) * a_ref[...]
    q_ref[...] = jnp.dot(hg, wq_ref[...], preferred_element_type=_f32) * (1.0 / 4.0)
    k_ref[...] = jnp.dot(hg, wk_ref[...], preferred_element_type=_f32)
    v_ref[...] = jnp.dot(hg, wv_ref[...], preferred_element_type=_f32)

  w_spec = pl.BlockSpec((_D, _D), lambda i: (0, 0))
  return pl.pallas_call(
      body,
      grid=(_N // T,),
      in_specs=[pl.BlockSpec((T, _D), lambda i: (i, 0)),
                pl.BlockSpec((T, _D), lambda i: (i, 0)),
                pl.BlockSpec((1, 1), lambda i: (0, 0)),
                pl.BlockSpec((T, 1), lambda i: (i, 0)),
                w_spec, w_spec, w_spec],
      out_specs=[pl.BlockSpec((T, _D), lambda i: (i, 0))] * 3,
      out_shape=[jax.ShapeDtypeStruct((_N, _D), _f32)] * 3,
  )(oi, of, p2, attr, wq, wk, wv)


def _bias_tc(len2, ew1, eb1, ew2):
  T = 3200

  def body(l_ref, w1_ref, b1_ref, w2_ref, o_ref):
    ln = jnp.sqrt(l_ref[...])
    feat = jnp.tanh(ln * w1_ref[...] + b1_ref[...])
    o_ref[...] = jnp.dot(feat, w2_ref[...], preferred_element_type=_f32)

  return pl.pallas_call(
      body,
      grid=(_E // T,),
      in_specs=[pl.BlockSpec((T, 1), lambda i: (i, 0)),
                pl.BlockSpec((1, _FC), lambda i: (0, 0)),
                pl.BlockSpec((1, _FC), lambda i: (0, 0)),
                pl.BlockSpec((_FC, 16), lambda i: (0, 0))],
      out_specs=pl.BlockSpec((T, 16), lambda i: (i, 0)),
      out_shape=jax.ShapeDtypeStruct((_E, 16), _f32),
  )(len2, ew1, eb1, jnp.pad(ew2, ((0, 0), (0, 8))))


def _finish_tc(parts, wo, expand):
  T = 1000
  parts3 = parts.reshape(_NC, _ANR, _AW)
  den_a = parts3[0, _NPAD:_NPAD + _N // 16, :].reshape(-1, _H)
  den_b = parts3[1, _NPAD:_NPAD + _N // 16, :].reshape(-1, _H)

  def body(p_ref, da_ref, db_ref, wo_ref, ex_ref, o_ref):
    num = p_ref[0] + p_ref[1]
    den = da_ref[...] + db_ref[...]
    rec = 1.0 / (den + 1e-9)
    rec128 = jnp.dot(rec, ex_ref[...], preferred_element_type=_f32)
    o_ref[...] = jnp.dot(num * rec128, wo_ref[...], preferred_element_type=_f32)

  return pl.pallas_call(
      body,
      grid=(_N // T,),
      in_specs=[pl.BlockSpec((_NC, T, _AW), lambda i: (0, i, 0)),
                pl.BlockSpec((T, _H), lambda i: (i, 0)),
                pl.BlockSpec((T, _H), lambda i: (i, 0)),
                pl.BlockSpec((_D, _D), lambda i: (0, 0)),
                pl.BlockSpec((_H, _D), lambda i: (0, 0))],
      out_specs=pl.BlockSpec((T, _D), lambda i: (i, 0)),
      out_shape=jax.ShapeDtypeStruct((_N, _D), _f32),
  )(parts3, den_a, den_b, wo, expand)


def _block_pre(pos, src, dst, prm):
  ln = _edge_len_sc(pos[:, 0], pos[:, 1], pos[:, 2], src, dst)
  return _bias_tc(ln.reshape(_E, 1), prm["ew1"], prm["eb1"].reshape(1, _FC),
                  prm["ew2"])


def kernel(x, node_attr, pos, edge_index, x_final_state, pos_final_state,
           edge_index_final_state, pos_interpolated_transition_state,
           edge_index_interpolated_transition_state, batch, p,
           params_init, params_final, params_ts):
  del batch
  expand = jnp.asarray(np.kron(np.eye(_H), np.ones((1, _DH))), dtype=_f32)

  src1, dst1 = edge_index[0], edge_index[1]
  src2, dst2 = edge_index_final_state[0], edge_index_final_state[1]
  src3, dst3 = (edge_index_interpolated_transition_state[0],
                edge_index_interpolated_transition_state[1])

  eb1 = _block_pre(pos, src1, dst1, params_init)
  eb2 = _block_pre(pos_final_state, src2, dst2, params_final)
  eb3 = _block_pre(pos_interpolated_transition_state, src3, dst3, params_ts)

  q1, k1, v1 = _qkv_tc(x, node_attr, params_init["Wq"], params_init["Wk"],
                       params_init["Wv"])
  parts1 = _edge_pass_sc(q1, k1, v1, src1, dst1, eb1)
  out1 = _finish_tc(parts1, params_init["Wo"], expand)

  q2, k2, v2 = _qkv_tc(x_final_state, node_attr, params_final["Wq"],
                       params_final["Wk"], params_final["Wv"])
  parts2 = _edge_pass_sc(q2, k2, v2, src2, dst2, eb2)
  out2 = _finish_tc(parts2, params_final["Wo"], expand)

  q3, k3, v3 = _qkv3_tc(out1, out2, p.reshape(1, 1), node_attr,
                        params_ts["Wq"], params_ts["Wk"], params_ts["Wv"])
  parts3 = _edge_pass_sc(q3, k3, v3, src3, dst3, eb3)
  return _finish_tc(parts3, params_ts["Wo"], expand)
